# baseline (device time: 20405 ns/iter reference)
import jax
import jax.numpy as jnp
from jax import lax
from jax.experimental import pallas as pl
from jax.experimental.pallas import tpu as pltpu

N_CHUNKS = 4


def kernel(Q, K, V):
    b, sq, h, d = Q.shape
    skv = K.shape[1]
    kc = skv // N_CHUNKS
    scale = d ** -0.5

    Kt = jnp.transpose(K, (0, 2, 3, 1))
    Vt = jnp.transpose(V, (0, 2, 3, 1))

    def body(q_ref, kt_ref, vt_ref, o_ref,
             acc_o, acc_l, send_buf, recv_buf, send_sem, recv_sem):
        step = pl.program_id(0)

        q = q_ref[:, 0, :, :]
        kt = kt_ref[...]
        vt = vt_ref[...]

        s = jnp.sum(q[..., None] * kt, axis=2) * scale
        p = jnp.exp(s)
        l_c = jnp.sum(p, axis=-1)
        o_c = jnp.sum(p[:, :, None, :] * vt, axis=-1)

        @pl.when(step == 0)
        def _():
            acc_o[...] = o_c
            acc_l[...] = l_c

        @pl.when(step != 0)
        def _():
            acc_o[...] += o_c
            acc_l[...] += l_c

        @pl.when(step == N_CHUNKS - 1)
        def _():
            my_x = lax.axis_index("x")
            my_y = lax.axis_index("y")
            my_z = lax.axis_index("z")
            partner = (my_x, 1 - my_y, my_z)

            barrier_sem = pltpu.get_barrier_semaphore()
            pl.semaphore_signal(
                barrier_sem, inc=1, device_id=partner,
                device_id_type=pl.DeviceIdType.MESH,
            )
            pl.semaphore_wait(barrier_sem, 1)

            send_buf[0, :, :, :] = acc_o[...]
            send_buf[1, :, :, :] = jnp.broadcast_to(
                acc_l[...][:, :, None], (b, h, d))

            rdma = pltpu.make_async_remote_copy(
                src_ref=send_buf,
                dst_ref=recv_buf,
                send_sem=send_sem,
                recv_sem=recv_sem,
                device_id=partner,
                device_id_type=pl.DeviceIdType.MESH,
            )
            rdma.start()
            rdma.wait()

            o_tot = send_buf[0, :, :, :] + recv_buf[0, :, :, :]
            l_tot = send_buf[1, :, :, :] + recv_buf[1, :, :, :]
            o_ref[:, 0, :, :] = o_tot / l_tot

    return pl.pallas_call(
        body,
        grid=(N_CHUNKS,),
        out_shape=jax.ShapeDtypeStruct((b, sq, h, d), jnp.float32),
        in_specs=[
            pl.BlockSpec((b, sq, h, d), lambda i: (0, 0, 0, 0)),
            pl.BlockSpec((b, h, d, kc), lambda i: (0, 0, 0, i)),
            pl.BlockSpec((b, h, d, kc), lambda i: (0, 0, 0, i)),
        ],
        out_specs=pl.BlockSpec((b, sq, h, d), lambda i: (0, 0, 0, 0)),
        scratch_shapes=[
            pltpu.VMEM((b, h, d), jnp.float32),
            pltpu.VMEM((b, h), jnp.float32),
            pltpu.VMEM((2, b, h, d), jnp.float32),
            pltpu.VMEM((2, b, h, d), jnp.float32),
            pltpu.SemaphoreType.DMA,
            pltpu.SemaphoreType.DMA,
        ],
        compiler_params=pltpu.CompilerParams(
            collective_id=0,
            dimension_semantics=("arbitrary",),
        ),
    )(Q, Kt, Vt)


# device time: 18818 ns/iter; 1.0843x vs baseline; 1.0843x over previous
import jax
import jax.numpy as jnp
from jax import lax
from jax.experimental import pallas as pl
from jax.experimental.pallas import tpu as pltpu

N_CHUNKS = 4


def kernel(Q, K, V):
    b, sq, h, d = Q.shape
    skv = K.shape[1]
    kc = skv // N_CHUNKS
    scale = d ** -0.5

    Kt = jnp.transpose(K, (0, 2, 3, 1))
    Vt = jnp.transpose(V, (0, 2, 3, 1))

    def body(q_ref, kt_hbm, vt_hbm, o_ref,
             kbuf, vbuf, send_buf, recv_buf, ksem, vsem, send_sem, recv_sem):
        def chunk_copies(c, slot):
            cp_k = pltpu.make_async_copy(
                kt_hbm.at[:, :, :, pl.ds(c * kc, kc)], kbuf.at[slot],
                ksem.at[slot])
            cp_v = pltpu.make_async_copy(
                vt_hbm.at[:, :, :, pl.ds(c * kc, kc)], vbuf.at[slot],
                vsem.at[slot])
            return cp_k, cp_v

        k0, v0 = chunk_copies(0, 0)
        k0.start()
        v0.start()

        q = q_ref[:, 0, :, :]
        acc_o = None
        acc_l = None
        pending = (k0, v0)
        for c in range(N_CHUNKS):
            slot = c % 2
            if c + 1 < N_CHUNKS:
                nxt = chunk_copies(c + 1, 1 - slot)
                nxt[0].start()
                nxt[1].start()
            pending[0].wait()
            pending[1].wait()
            if c + 1 < N_CHUNKS:
                pending = nxt

            kt = kbuf[slot]
            vt = vbuf[slot]
            s = jnp.sum(q[..., None] * kt, axis=2) * scale
            p = jnp.exp(s)
            l_c = jnp.sum(p, axis=-1)
            o_c = jnp.sum(p[:, :, None, :] * vt, axis=-1)
            acc_o = o_c if acc_o is None else acc_o + o_c
            acc_l = l_c if acc_l is None else acc_l + l_c

        my_x = lax.axis_index("x")
        my_y = lax.axis_index("y")
        my_z = lax.axis_index("z")
        partner = (my_x, 1 - my_y, my_z)

        barrier_sem = pltpu.get_barrier_semaphore()
        pl.semaphore_signal(
            barrier_sem, inc=1, device_id=partner,
            device_id_type=pl.DeviceIdType.MESH,
        )
        pl.semaphore_wait(barrier_sem, 1)

        send_buf[0, :, :, :] = acc_o
        send_buf[1, :, :, :] = jnp.broadcast_to(acc_l[:, :, None], (b, h, d))

        rdma = pltpu.make_async_remote_copy(
            src_ref=send_buf,
            dst_ref=recv_buf,
            send_sem=send_sem,
            recv_sem=recv_sem,
            device_id=partner,
            device_id_type=pl.DeviceIdType.MESH,
        )
        rdma.start()
        rdma.wait()

        o_tot = send_buf[0, :, :, :] + recv_buf[0, :, :, :]
        l_tot = send_buf[1, :, :, :] + recv_buf[1, :, :, :]
        o_ref[:, 0, :, :] = o_tot / l_tot

    return pl.pallas_call(
        body,
        out_shape=jax.ShapeDtypeStruct((b, sq, h, d), jnp.float32),
        in_specs=[
            pl.BlockSpec(memory_space=pltpu.VMEM),
            pl.BlockSpec(memory_space=pl.ANY),
            pl.BlockSpec(memory_space=pl.ANY),
        ],
        out_specs=pl.BlockSpec(memory_space=pltpu.VMEM),
        scratch_shapes=[
            pltpu.VMEM((2, b, h, d, kc), jnp.float32),
            pltpu.VMEM((2, b, h, d, kc), jnp.float32),
            pltpu.VMEM((2, b, h, d), jnp.float32),
            pltpu.VMEM((2, b, h, d), jnp.float32),
            pltpu.SemaphoreType.DMA((2,)),
            pltpu.SemaphoreType.DMA((2,)),
            pltpu.SemaphoreType.DMA,
            pltpu.SemaphoreType.DMA,
        ],
        compiler_params=pltpu.CompilerParams(collective_id=0),
    )(Q, Kt, Vt)


# device time: 14639 ns/iter; 1.3939x vs baseline; 1.2855x over previous
import jax
import jax.numpy as jnp
from jax import lax
from jax.experimental import pallas as pl
from jax.experimental.pallas import tpu as pltpu


def kernel(Q, K, V):
    b, sq, h, d = Q.shape
    skv = K.shape[1]
    scale = d ** -0.5

    Kt = jnp.transpose(K, (0, 2, 3, 1))
    Vt = jnp.transpose(V, (0, 2, 3, 1))

    def body(q_ref, kt_ref, vt_ref, o_ref,
             send_buf, recv_buf, send_sem, recv_sem):
        my_x = lax.axis_index("x")
        my_y = lax.axis_index("y")
        my_z = lax.axis_index("z")
        partner = (my_x, 1 - my_y, my_z)

        barrier_sem = pltpu.get_barrier_semaphore()
        pl.semaphore_signal(
            barrier_sem, inc=1, device_id=partner,
            device_id_type=pl.DeviceIdType.MESH,
        )
        pl.semaphore_wait(barrier_sem, 1)

        q = q_ref[:, 0, :, :]
        kt = kt_ref[...]
        vt = vt_ref[...]

        l_rows = []
        o_rows = []
        for bi in range(b):
            s_b = lax.dot_general(
                q[bi], kt[bi],
                dimension_numbers=(((1,), (1,)), ((0,), (0,))),
                preferred_element_type=jnp.float32,
            ) * scale
            p_b = jnp.exp(s_b)
            l_rows.append(jnp.sum(p_b, axis=-1))
            o_rows.append(lax.dot_general(
                p_b, vt[bi],
                dimension_numbers=(((1,), (2,)), ((0,), (0,))),
                preferred_element_type=jnp.float32,
            ))
        l_c = jnp.stack(l_rows)
        o_c = jnp.stack(o_rows)

        send_buf[0, :, :, :] = o_c
        send_buf[1, :, :, :] = jnp.broadcast_to(l_c[:, :, None], (b, h, d))

        rdma = pltpu.make_async_remote_copy(
            src_ref=send_buf,
            dst_ref=recv_buf,
            send_sem=send_sem,
            recv_sem=recv_sem,
            device_id=partner,
            device_id_type=pl.DeviceIdType.MESH,
        )
        rdma.start()
        rdma.wait()

        o_tot = send_buf[0, :, :, :] + recv_buf[0, :, :, :]
        l_tot = send_buf[1, :, :, :] + recv_buf[1, :, :, :]
        o_ref[:, 0, :, :] = o_tot / l_tot

    return pl.pallas_call(
        body,
        out_shape=jax.ShapeDtypeStruct((b, sq, h, d), jnp.float32),
        in_specs=[
            pl.BlockSpec(memory_space=pltpu.VMEM),
            pl.BlockSpec(memory_space=pltpu.VMEM),
            pl.BlockSpec(memory_space=pltpu.VMEM),
        ],
        out_specs=pl.BlockSpec(memory_space=pltpu.VMEM),
        scratch_shapes=[
            pltpu.VMEM((2, b, h, d), jnp.float32),
            pltpu.VMEM((2, b, h, d), jnp.float32),
            pltpu.SemaphoreType.DMA,
            pltpu.SemaphoreType.DMA,
        ],
        compiler_params=pltpu.CompilerParams(collective_id=0),
    )(Q, Kt, Vt)


# device time: 13145 ns/iter; 1.5523x vs baseline; 1.1137x over previous
import jax
import jax.numpy as jnp
from jax import lax
from jax.experimental import pallas as pl
from jax.experimental.pallas import tpu as pltpu

P = 4


def kernel(Q, K, V):
    b, sq, h, d = Q.shape
    skv = K.shape[1]
    scale = d ** -0.5

    Kt = jnp.transpose(K, (0, 2, 3, 1))
    Vt = jnp.transpose(V, (0, 2, 3, 1))

    xk = skv // 2
    x_idx = lax.axis_index("x")
    Kt = lax.dynamic_slice_in_dim(Kt, x_idx * xk, xk, axis=3)
    Vt = lax.dynamic_slice_in_dim(Vt, x_idx * xk, xk, axis=3)

    def body(q_ref, kt_ref, vt_ref, o_ref,
             send_buf, recv_bufs, local_sem, send_sems, recv_sems):
        my_x = lax.axis_index("x")
        my_y = lax.axis_index("y")
        my_z = lax.axis_index("z")
        pid = my_x * 2 + my_y

        barrier_sem = pltpu.get_barrier_semaphore()
        for off in range(1, P):
            pfid = (pid + off) % P
            pl.semaphore_signal(
                barrier_sem, inc=1,
                device_id=(pfid // 2, pfid % 2, my_z),
                device_id_type=pl.DeviceIdType.MESH,
            )
        pl.semaphore_wait(barrier_sem, P - 1)

        q = q_ref[:, 0, :, :]
        kt = kt_ref[...]
        vt = vt_ref[...]

        s = jnp.sum(q[..., None] * kt, axis=2) * scale
        p = jnp.exp(s)
        l_c = jnp.sum(p, axis=-1)
        o_c = jnp.sum(p[:, :, None, :] * vt, axis=-1)

        send_buf[0, :, :, :] = o_c
        send_buf[1, :, :, :] = jnp.broadcast_to(l_c[:, :, None], (b, h, d))

        sends = []
        for off in range(1, P):
            pfid = (pid + off) % P
            rd = pltpu.make_async_remote_copy(
                src_ref=send_buf,
                dst_ref=recv_bufs.at[pid],
                send_sem=send_sems.at[pfid],
                recv_sem=recv_sems.at[pid],
                device_id=(pfid // 2, pfid % 2, my_z),
                device_id_type=pl.DeviceIdType.MESH,
            )
            rd.start()
            sends.append(rd)
        cp_self = pltpu.make_async_copy(send_buf, recv_bufs.at[pid], local_sem)
        cp_self.start()

        for off in range(1, P):
            pfid = (pid + off) % P
            pltpu.make_async_remote_copy(
                src_ref=send_buf,
                dst_ref=recv_bufs.at[pfid],
                send_sem=send_sems.at[pfid],
                recv_sem=recv_sems.at[pfid],
                device_id=(pfid // 2, pfid % 2, my_z),
                device_id_type=pl.DeviceIdType.MESH,
            ).wait_recv()
        cp_self.wait()

        tot = jnp.sum(recv_bufs[...], axis=0)
        o_ref[:, 0, :, :] = tot[0] / tot[1]

        for rd in sends:
            rd.wait_send()

    return pl.pallas_call(
        body,
        out_shape=jax.ShapeDtypeStruct((b, sq, h, d), jnp.float32),
        in_specs=[
            pl.BlockSpec(memory_space=pltpu.VMEM),
            pl.BlockSpec(memory_space=pltpu.VMEM),
            pl.BlockSpec(memory_space=pltpu.VMEM),
        ],
        out_specs=pl.BlockSpec(memory_space=pltpu.VMEM),
        scratch_shapes=[
            pltpu.VMEM((2, b, h, d), jnp.float32),
            pltpu.VMEM((P, 2, b, h, d), jnp.float32),
            pltpu.SemaphoreType.DMA,
            pltpu.SemaphoreType.DMA((P,)),
            pltpu.SemaphoreType.DMA((P,)),
        ],
        compiler_params=pltpu.CompilerParams(collective_id=0),
    )(Q, Kt, Vt)
